# baseline (speedup 1.0000x reference)
import functools

import jax
import jax.numpy as jnp
from jax import lax
from jax.experimental import pallas as pl
from jax.experimental.pallas import tpu as pltpu
from jax.experimental.pallas import tpu_sc as plsc

_NUM_CORES = 2
_NUM_SUBCORES = 16
_NUM_WORKERS = _NUM_CORES * _NUM_SUBCORES
_GATHER_CHUNK = 128


def _make_sc_gather(vocab: int, dim: int, batch: int):
  assert batch % (8 * _NUM_WORKERS) == 0
  b_per_w = batch // _NUM_WORKERS
  n_chunks = b_per_w // _GATHER_CHUNK
  assert n_chunks * _GATHER_CHUNK == b_per_w
  mesh = plsc.VectorSubcoreMesh(core_axis_name="c", subcore_axis_name="s")

  @functools.partial(
      pl.kernel,
      mesh=mesh,
      out_type=jax.ShapeDtypeStruct((batch, dim), jnp.float32),
      scratch_types=[
          pltpu.VMEM((b_per_w,), jnp.int32),
          pltpu.VMEM((b_per_w, dim), jnp.float32),
          pltpu.SemaphoreType.DMA,
      ],
      compiler_params=pltpu.CompilerParams(use_tc_tiling_on_sc=False),
  )
  def gather(table_hbm, idx_hbm, out_hbm, idx_v, rows_v, sem):
    wid = lax.axis_index("s") * _NUM_CORES + lax.axis_index("c")
    base = wid * b_per_w
    pltpu.sync_copy(idx_hbm.at[pl.ds(base, b_per_w)], idx_v)
    copies = []
    for j in range(n_chunks):
      copies.append(
          pltpu.make_async_copy(
              table_hbm.at[idx_v.at[pl.ds(j * _GATHER_CHUNK, _GATHER_CHUNK)]],
              rows_v.at[pl.ds(j * _GATHER_CHUNK, _GATHER_CHUNK)],
              sem,
          )
      )
      copies[-1].start()
    for c in copies:
      c.wait()
    pltpu.sync_copy(rows_v, out_hbm.at[pl.ds(base, b_per_w)])

  return gather


def _mlp_body(h_ref, w1_ref, b1_ref, w2_ref, b2_ref, o_ref):
  h = h_ref[...]
  z1 = jax.lax.dot_general(
      h, w1_ref[...], (((1,), (1,)), ((), ())),
      preferred_element_type=jnp.float32) + b1_ref[...]
  o_ref[...] = jax.lax.dot_general(
      z1, w2_ref[...], (((1,), (1,)), ((), ())),
      preferred_element_type=jnp.float32) + b2_ref[...]


def _tc_mlp(h, W1, b1, W2, b2, block: int = 2048):
  batch, dim = h.shape
  out_dim = W2.shape[0]
  grid = (batch // block,)
  return pl.pallas_call(
      _mlp_body,
      grid=grid,
      in_specs=[
          pl.BlockSpec((block, dim), lambda i: (i, 0)),
          pl.BlockSpec((dim, dim), lambda i: (0, 0)),
          pl.BlockSpec((1, dim), lambda i: (0, 0)),
          pl.BlockSpec((out_dim, dim), lambda i: (0, 0)),
          pl.BlockSpec((1, out_dim), lambda i: (0, 0)),
      ],
      out_specs=pl.BlockSpec((block, out_dim), lambda i: (i, 0)),
      out_shape=jax.ShapeDtypeStruct((batch, out_dim), jnp.float32),
  )(h, W1, b1.reshape(1, dim), W2, b2.reshape(1, out_dim))


@jax.jit
def kernel(x, table, W1, b1, W2, b2):
  vocab, dim = table.shape
  (batch,) = x.shape
  h = _make_sc_gather(vocab, dim, batch)(table, x)
  return _tc_mlp(h, W1, b1, W2, b2)
